# Initial kernel scaffold; baseline (speedup 1.0000x reference)
import jax
import jax.numpy as jnp
from jax.experimental import pallas as pl


def kernel(feature_tokens, ln_gamma, ln_beta, W1, b1, W2, b2):
    raise NotImplementedError("write your pallas kernel here")


# speedup vs baseline: 2.1386x; 2.1386x over previous
import functools

import jax
import jax.numpy as jnp
from jax.experimental import pallas as pl
from jax.experimental.pallas import tpu as pltpu

_CHUNK = 8


def _readout_kernel(x_ref, a_ref, gw_ref, bw_ref, w2_ref, b2_ref,
                    sum_ref, wt_ref):
    a = a_ref[...]
    gw = gw_ref[...]
    bw = bw_ref[...]
    w2 = w2_ref[...]
    b2 = b2_ref[...]
    h_dim = gw.shape[0]
    d = a.shape[0]
    inv_d = 1.0 / d
    ones_row = jnp.ones((1, d), dtype=jnp.float32)
    for c in range(_CHUNK):
        x = x_ref[c]
        q = jax.lax.dot_general(a, x, (((0,), (1,)), ((), ())),
                                preferred_element_type=jnp.float32)
        s2 = jax.lax.dot_general(ones_row, x * x, (((1,), (1,)), ((), ())),
                                 preferred_element_type=jnp.float32)
        p = q[:h_dim, :]
        s1 = q[h_dim:h_dim + 1, :]
        mu = s1 * inv_d
        var = s2 * inv_d - mu * mu
        rs = jax.lax.rsqrt(var + 1e-5)
        hpre = rs * p - (rs * mu) * gw + bw
        h = 0.5 * hpre * (1.0 + jax.lax.erf(hpre * 0.7071067811865476))
        logits = jax.lax.dot_general(w2, h, (((1,), (0,)), ((), ())),
                                     preferred_element_type=jnp.float32) + b2
        m = jnp.max(logits, axis=1, keepdims=True)
        e = jnp.exp(logits - m)
        denom = jnp.sum(e, axis=1, keepdims=True)
        w = e / denom
        summ = jax.lax.dot_general(w, x, (((1,), (0,)), ((), ())),
                                   preferred_element_type=jnp.float32)
        sum_ref[0, pl.ds(c, 1), :] = summ
        wt_ref[0, pl.ds(c, 1), :] = w


@functools.partial(jax.jit, static_argnames=())
def kernel(feature_tokens, ln_gamma, ln_beta, W1, b1, W2, b2):
    B, N, D = feature_tokens.shape
    H = W1.shape[1]
    f32 = jnp.float32
    w1g = W1 * ln_gamma[:, None]
    a = jnp.concatenate([w1g, jnp.ones((D, 1), f32)], axis=1)
    gw = (ln_gamma @ W1)[:, None]
    bw = (ln_beta @ W1 + b1)[:, None]
    w2r = W2[None, :]
    b2s = jnp.reshape(b2, (1, 1))

    grid = (B // _CHUNK,)
    summary3, weights3 = pl.pallas_call(
        _readout_kernel,
        out_shape=(
            jax.ShapeDtypeStruct((B // _CHUNK, _CHUNK, D), f32),
            jax.ShapeDtypeStruct((B // _CHUNK, _CHUNK, N), f32),
        ),
        grid=grid,
        in_specs=[
            pl.BlockSpec((_CHUNK, N, D), lambda i: (i, 0, 0)),
            pl.BlockSpec((D, H + 1), lambda i: (0, 0)),
            pl.BlockSpec((H, 1), lambda i: (0, 0)),
            pl.BlockSpec((H, 1), lambda i: (0, 0)),
            pl.BlockSpec((1, H), lambda i: (0, 0)),
            pl.BlockSpec((1, 1), lambda i: (0, 0)),
        ],
        out_specs=(
            pl.BlockSpec((1, _CHUNK, D), lambda i: (i, 0, 0)),
            pl.BlockSpec((1, _CHUNK, N), lambda i: (i, 0, 0)),
        ),
        compiler_params=pltpu.CompilerParams(
            dimension_semantics=("parallel",),
        ),
        name="token_attentive_readout",
    )(feature_tokens, a, gw, bw, w2r, b2s)
    return summary3.reshape(B, D), weights3.reshape(B, N)


# speedup vs baseline: 2.7990x; 1.3088x over previous
import jax
import jax.numpy as jnp
from jax.experimental import pallas as pl
from jax.experimental.pallas import tpu as pltpu

_CHUNK = 8


def _readout_kernel(x_ref, a_ref, gw_ref, bw_ref, w2_ref, b2_ref,
                    sum_ref, wt_ref):
    a = a_ref[...]
    gw = gw_ref[...]
    bw = bw_ref[...]
    w2 = w2_ref[...]
    b2 = b2_ref[...]
    h_dim = gw.shape[0]
    d = a.shape[0]
    inv_d = 1.0 / d
    ones_row = jnp.ones((1, d), dtype=jnp.float32)

    logits_rows = []
    for c in range(_CHUNK):
        x = x_ref[c]
        q = jax.lax.dot_general(a, x, (((0,), (1,)), ((), ())),
                                preferred_element_type=jnp.float32)
        s2 = jax.lax.dot_general(ones_row, x * x, (((1,), (1,)), ((), ())),
                                 preferred_element_type=jnp.float32)
        p = q[:h_dim, :]
        s1 = q[h_dim:h_dim + 1, :]
        mu = s1 * inv_d
        var = s2 * inv_d - mu * mu
        rs = jax.lax.rsqrt(var + 1e-5)
        hpre = rs * p - (rs * mu) * gw + bw
        h = 0.5 * hpre * (1.0 + jax.lax.erf(hpre * 0.7071067811865476))
        logits_rows.append(
            jax.lax.dot_general(w2, h, (((1,), (0,)), ((), ())),
                                preferred_element_type=jnp.float32) + b2)
    rows = jnp.concatenate(logits_rows, axis=0)
    m = jnp.max(rows, axis=1, keepdims=True)
    e = jnp.exp(rows - m)
    denom = jnp.sum(e, axis=1, keepdims=True)
    w = e / denom
    wt_ref[0] = w
    for c in range(_CHUNK):
        summ = jax.lax.dot_general(w[c:c + 1, :], x_ref[c],
                                   (((1,), (0,)), ((), ())),
                                   preferred_element_type=jnp.float32)
        sum_ref[0, pl.ds(c, 1), :] = summ


def kernel(feature_tokens, ln_gamma, ln_beta, W1, b1, W2, b2):
    B, N, D = feature_tokens.shape
    H = W1.shape[1]
    f32 = jnp.float32
    w1g = W1 * ln_gamma[:, None]
    a = jnp.concatenate([w1g, jnp.ones((D, 1), f32)], axis=1)
    gw = (ln_gamma @ W1)[:, None]
    bw = (ln_beta @ W1 + b1)[:, None]
    w2r = W2[None, :]
    b2s = jnp.reshape(b2, (1, 1))

    grid = (B // _CHUNK,)
    summary3, weights3 = pl.pallas_call(
        _readout_kernel,
        out_shape=(
            jax.ShapeDtypeStruct((B // _CHUNK, _CHUNK, D), f32),
            jax.ShapeDtypeStruct((B // _CHUNK, _CHUNK, N), f32),
        ),
        grid=grid,
        in_specs=[
            pl.BlockSpec((_CHUNK, N, D), lambda i: (i, 0, 0)),
            pl.BlockSpec((D, H + 1), lambda i: (0, 0)),
            pl.BlockSpec((H, 1), lambda i: (0, 0)),
            pl.BlockSpec((H, 1), lambda i: (0, 0)),
            pl.BlockSpec((1, H), lambda i: (0, 0)),
            pl.BlockSpec((1, 1), lambda i: (0, 0)),
        ],
        out_specs=(
            pl.BlockSpec((1, _CHUNK, D), lambda i: (i, 0, 0)),
            pl.BlockSpec((1, _CHUNK, N), lambda i: (i, 0, 0)),
        ),
        compiler_params=pltpu.CompilerParams(
            dimension_semantics=("parallel",),
        ),
        name="token_attentive_readout",
    )(feature_tokens, a, gw, bw, w2r, b2s)
    return summary3.reshape(B, D), weights3.reshape(B, N)


# speedup vs baseline: 3.6847x; 1.3164x over previous
import jax
import jax.numpy as jnp
from jax.experimental import pallas as pl
from jax.experimental.pallas import tpu as pltpu

_CHUNK = 8
_HALF = _CHUNK


def _readout_kernel(x_ref, a_ref, gw_ref, bw_ref, w2_ref, b2_ref,
                    sum_ref, wt_ref):
    a = a_ref[...]
    gw = gw_ref[...]
    bw = bw_ref[...]
    w2 = w2_ref[...]
    b2 = b2_ref[...]
    h_dim = gw.shape[0]
    d = a.shape[0]
    inv_d = 1.0 / d
    ones_row = jnp.ones((1, d), dtype=jnp.bfloat16)

    def matmuls(c):
        xb = x_ref[c].astype(jnp.bfloat16)
        q = jax.lax.dot_general(a, xb, (((0,), (1,)), ((), ())),
                                preferred_element_type=jnp.float32)
        s2 = jax.lax.dot_general(ones_row, xb * xb, (((1,), (1,)), ((), ())),
                                 preferred_element_type=jnp.float32)
        return xb, q, s2

    def scorer_tail(q, s2):
        p = q[:h_dim, :]
        s1 = q[h_dim:h_dim + 1, :]
        mu = s1 * inv_d
        var = s2 * inv_d - mu * mu
        rs = jax.lax.rsqrt(var + 1e-5)
        hpre = rs * p - (rs * mu) * gw + bw
        h = hpre * (0.5 * jax.lax.erf(hpre * 0.7071067811865476) + 0.5)
        return (jax.lax.dot_general(w2, h.astype(jnp.bfloat16),
                                    (((1,), (0,)), ((), ())),
                                    preferred_element_type=jnp.float32)
                + b2)

    def readout(c, logit_row, xb):
        m = jnp.max(logit_row, axis=1, keepdims=True)
        e = jnp.exp(logit_row - m)
        denom = jnp.sum(e, axis=1, keepdims=True)
        w = e / denom
        wt_ref[0, pl.ds(c, 1), :] = w
        summ = jax.lax.dot_general(w.astype(jnp.bfloat16), xb,
                                   (((1,), (0,)), ((), ())),
                                   preferred_element_type=jnp.float32)
        sum_ref[0, pl.ds(c, 1), :] = summ

    xbs, staged, tailq = [], None, []
    for c in range(_CHUNK):
        xb, q, s2 = matmuls(c)
        xbs.append(xb)
        if staged is not None:
            tailq.append(scorer_tail(*staged))
        staged = (q, s2)
        if len(tailq) > 1:
            readout(c - 2, tailq.pop(0), xbs[c - 2])
    tailq.append(scorer_tail(*staged))
    readout(_CHUNK - 2, tailq.pop(0), xbs[_CHUNK - 2])
    readout(_CHUNK - 1, tailq.pop(0), xbs[_CHUNK - 1])


def kernel(feature_tokens, ln_gamma, ln_beta, W1, b1, W2, b2):
    B, N, D = feature_tokens.shape
    H = W1.shape[1]
    f32 = jnp.float32
    w1g = W1 * ln_gamma[:, None]
    a = jnp.concatenate([w1g, jnp.ones((D, 1), f32)],
                        axis=1).astype(jnp.bfloat16)
    gw = (ln_gamma @ W1)[:, None]
    bw = (ln_beta @ W1 + b1)[:, None]
    w2r = W2[None, :].astype(jnp.bfloat16)
    b2s = jnp.reshape(b2, (1, 1))

    grid = (B // _CHUNK,)
    summary3, weights3 = pl.pallas_call(
        _readout_kernel,
        out_shape=(
            jax.ShapeDtypeStruct((B // _CHUNK, _CHUNK, D), f32),
            jax.ShapeDtypeStruct((B // _CHUNK, _CHUNK, N), f32),
        ),
        grid=grid,
        in_specs=[
            pl.BlockSpec((_CHUNK, N, D), lambda i: (i, 0, 0)),
            pl.BlockSpec((D, H + 1), lambda i: (0, 0)),
            pl.BlockSpec((H, 1), lambda i: (0, 0)),
            pl.BlockSpec((H, 1), lambda i: (0, 0)),
            pl.BlockSpec((1, H), lambda i: (0, 0)),
            pl.BlockSpec((1, 1), lambda i: (0, 0)),
        ],
        out_specs=(
            pl.BlockSpec((1, _CHUNK, D), lambda i: (i, 0, 0)),
            pl.BlockSpec((1, _CHUNK, N), lambda i: (i, 0, 0)),
        ),
        compiler_params=pltpu.CompilerParams(
            dimension_semantics=("parallel",),
        ),
        name="token_attentive_readout",
    )(feature_tokens, a, gw, bw, w2r, b2s)
    return summary3.reshape(B, D), weights3.reshape(B, N)


# speedup vs baseline: 4.4130x; 1.1977x over previous
import jax
import jax.numpy as jnp
from jax.experimental import pallas as pl
from jax.experimental.pallas import tpu as pltpu

_CHUNK = 16
_HALF = _CHUNK


def _readout_kernel(x_ref, a_ref, gw_ref, bw_ref, w2_ref, b2_ref,
                    sum_ref, wt_ref):
    a = a_ref[...]
    gw = gw_ref[...]
    bw = bw_ref[...]
    w2 = w2_ref[...]
    b2 = b2_ref[...]
    h_dim = gw.shape[0]
    d = a.shape[0]
    inv_d = 1.0 / d
    ones_f8 = jnp.ones((1, d), dtype=jnp.float8_e4m3fn)

    def matmuls(c):
        xb = x_ref[c].astype(jnp.bfloat16)
        q = jax.lax.dot_general(a, xb, (((0,), (1,)), ((), ())),
                                preferred_element_type=jnp.float32)
        x2 = (xb * xb).astype(jnp.float8_e4m3fn)
        s2 = jax.lax.dot_general(ones_f8, x2, (((1,), (1,)), ((), ())),
                                 preferred_element_type=jnp.float32)
        return xb, q, s2

    def scorer_tail(q, s2):
        p = q[:h_dim, :]
        s1 = q[h_dim:h_dim + 1, :]
        mu = s1 * inv_d
        var = s2 * inv_d - mu * mu
        rs = jax.lax.rsqrt(var + 1e-5)
        hpre = rs * p - (rs * mu) * gw + bw
        h = hpre * (0.5 * jax.lax.erf(hpre * 0.7071067811865476) + 0.5)
        return (jax.lax.dot_general(w2, h.astype(jnp.bfloat16),
                                    (((1,), (0,)), ((), ())),
                                    preferred_element_type=jnp.float32)
                + b2)

    def readout(c, logit_row, xb):
        m = jnp.max(logit_row, axis=1, keepdims=True)
        e = jnp.exp(logit_row - m)
        denom = jnp.sum(e, axis=1, keepdims=True)
        w = e / denom
        wt_ref[0, pl.ds(c, 1), :] = w
        summ = jax.lax.dot_general(w.astype(jnp.bfloat16), xb,
                                   (((1,), (0,)), ((), ())),
                                   preferred_element_type=jnp.float32)
        sum_ref[0, pl.ds(c, 1), :] = summ

    xbs, staged, tailq = [], None, []
    for c in range(_CHUNK):
        xb, q, s2 = matmuls(c)
        xbs.append(xb)
        if staged is not None:
            tailq.append(scorer_tail(*staged))
        staged = (q, s2)
        if len(tailq) > 1:
            readout(c - 2, tailq.pop(0), xbs[c - 2])
    tailq.append(scorer_tail(*staged))
    readout(_CHUNK - 2, tailq.pop(0), xbs[_CHUNK - 2])
    readout(_CHUNK - 1, tailq.pop(0), xbs[_CHUNK - 1])


def kernel(feature_tokens, ln_gamma, ln_beta, W1, b1, W2, b2):
    B, N, D = feature_tokens.shape
    H = W1.shape[1]
    f32 = jnp.float32
    w1g = W1 * ln_gamma[:, None]
    a = jnp.concatenate([w1g, jnp.ones((D, 1), f32)],
                        axis=1).astype(jnp.bfloat16)
    gw = (ln_gamma @ W1)[:, None]
    bw = (ln_beta @ W1 + b1)[:, None]
    w2r = W2[None, :].astype(jnp.bfloat16)
    b2s = jnp.reshape(b2, (1, 1))

    grid = (B // _CHUNK,)
    summary3, weights3 = pl.pallas_call(
        _readout_kernel,
        out_shape=(
            jax.ShapeDtypeStruct((B // _CHUNK, _CHUNK, D), f32),
            jax.ShapeDtypeStruct((B // _CHUNK, _CHUNK, N), f32),
        ),
        grid=grid,
        in_specs=[
            pl.BlockSpec((_CHUNK, N, D), lambda i: (i, 0, 0)),
            pl.BlockSpec((D, H + 1), lambda i: (0, 0)),
            pl.BlockSpec((H, 1), lambda i: (0, 0)),
            pl.BlockSpec((H, 1), lambda i: (0, 0)),
            pl.BlockSpec((1, H), lambda i: (0, 0)),
            pl.BlockSpec((1, 1), lambda i: (0, 0)),
        ],
        out_specs=(
            pl.BlockSpec((1, _CHUNK, D), lambda i: (i, 0, 0)),
            pl.BlockSpec((1, _CHUNK, N), lambda i: (i, 0, 0)),
        ),
        compiler_params=pltpu.CompilerParams(
            dimension_semantics=("parallel",),
            vmem_limit_bytes=52 * 1024 * 1024,
        ),
        name="token_attentive_readout",
    )(feature_tokens, a, gw, bw, w2r, b2s)
    return summary3.reshape(B, D), weights3.reshape(B, N)
